# Initial kernel scaffold; baseline (speedup 1.0000x reference)
import jax
import jax.numpy as jnp
from jax.experimental import pallas as pl


def kernel(inputs, table):
    raise NotImplementedError("write your pallas kernel here")


# speedup vs baseline: 2.4123x; 2.4123x over previous
import functools
import math

import jax
import jax.numpy as jnp
from jax import lax
from jax.experimental import pallas as pl
from jax.experimental.pallas import tpu as pltpu
from jax.experimental.pallas import tpu_sc as plsc

_CHUNK = 128


@functools.lru_cache(maxsize=None)
def _make_lookup(b_total: int, vocab: int, d: int):
    info = plsc.get_sparse_core_info()
    nc, ns, nl = info.num_cores, info.num_subcores, info.num_lanes
    nw = nc * ns
    assert d % nl == 0
    assert b_total % (nw * _CHUNK) == 0
    b_per_w = b_total // nw
    n_chunks = b_per_w // _CHUNK
    scale = jnp.float32(math.sqrt(float(d)))
    mesh = plsc.VectorSubcoreMesh(core_axis_name="c", subcore_axis_name="s")

    @functools.partial(
        pl.kernel,
        mesh=mesh,
        out_type=jax.ShapeDtypeStruct((b_total, d), jnp.float32),
        scratch_types=[
            pltpu.VMEM((n_chunks, _CHUNK), jnp.int32),
            pltpu.VMEM((_CHUNK, d), jnp.float32),
            pltpu.SemaphoreType.DMA,
        ],
    )
    def lookup(idx_hbm, table_hbm, out_hbm, idx_v, rows_v, sem):
        wid = lax.axis_index("s") * nc + lax.axis_index("c")
        base = wid * b_per_w
        pltpu.sync_copy(idx_hbm.at[wid], idx_v)

        def chunk_body(g, carry):
            pltpu.async_copy(table_hbm.at[idx_v.at[g]], rows_v, sem).wait()

            def row_body(i, c):
                for j in range(d // nl):
                    sl = pl.ds(j * nl, nl)
                    rows_v[i, sl] = rows_v[i, sl] * scale
                return c

            lax.fori_loop(0, _CHUNK, row_body, 0, unroll=2)

            pltpu.sync_copy(rows_v, out_hbm.at[pl.ds(base + g * _CHUNK, _CHUNK)])
            return carry

        lax.fori_loop(0, n_chunks, chunk_body, 0)

    return lookup


def kernel(inputs, table):
    b, s = inputs.shape
    vocab, d = table.shape
    b_total = b * s
    info = plsc.get_sparse_core_info()
    nw = info.num_cores * info.num_subcores
    n_chunks = b_total // (nw * _CHUNK)
    idx = inputs.reshape(nw, n_chunks, _CHUNK).astype(jnp.int32)
    out = _make_lookup(b_total, vocab, d)(idx, table)
    return out.reshape(b, s, d)


# speedup vs baseline: 2.8737x; 1.1913x over previous
import functools
import math

import jax
import jax.numpy as jnp
from jax import lax
from jax.experimental import pallas as pl
from jax.experimental.pallas import tpu as pltpu
from jax.experimental.pallas import tpu_sc as plsc

_CHUNK = 128


@functools.lru_cache(maxsize=None)
def _make_lookup(b_total: int, vocab: int, d: int):
    info = plsc.get_sparse_core_info()
    nc, ns, nl = info.num_cores, info.num_subcores, info.num_lanes
    nw = nc * ns
    assert d % nl == 0
    assert b_total % (nw * _CHUNK) == 0
    b_per_w = b_total // nw
    n_chunks = b_per_w // _CHUNK
    scale = jnp.float32(math.sqrt(float(d)))
    mesh = plsc.VectorSubcoreMesh(core_axis_name="c", subcore_axis_name="s")

    @functools.partial(
        pl.kernel,
        mesh=mesh,
        out_type=jax.ShapeDtypeStruct((b_total, d), jnp.float32),
        scratch_types=[
            pltpu.VMEM((n_chunks, _CHUNK), jnp.int32),
            pltpu.VMEM((_CHUNK, d), jnp.float32),
            pltpu.VMEM((_CHUNK, d), jnp.float32),
            pltpu.SemaphoreType.DMA,
            pltpu.SemaphoreType.DMA,
        ],
    )
    def lookup(idx_hbm, table_hbm, out_hbm, idx_v, rows0, rows1, sem0, sem1):
        wid = lax.axis_index("s") * nc + lax.axis_index("c")
        base = wid * b_per_w
        pltpu.sync_copy(idx_hbm.at[wid], idx_v)

        def start(g, buf, sem):
            pltpu.async_copy(table_hbm.at[idx_v.at[g]], buf, sem)

        def wait(buf, sem):
            pltpu.make_async_copy(table_hbm.at[pl.ds(0, _CHUNK)], buf, sem).wait()

        def scale_rows(buf):
            @plsc.parallel_loop(0, _CHUNK, 1, unroll=4)
            def _(i):
                for j in range(d // nl):
                    sl = pl.ds(j * nl, nl)
                    buf[i, sl] = buf[i, sl] * scale

        def emit(g, buf):
            pltpu.sync_copy(buf, out_hbm.at[pl.ds(base + g * _CHUNK, _CHUNK)])

        start(0, rows0, sem0)

        def pair_body(h, carry):
            g0 = 2 * h
            start(g0 + 1, rows1, sem1)
            wait(rows0, sem0)
            scale_rows(rows0)
            emit(g0, rows0)

            @pl.when(g0 + 2 < n_chunks)
            def _():
                start(g0 + 2, rows0, sem0)

            wait(rows1, sem1)
            scale_rows(rows1)
            emit(g0 + 1, rows1)
            return carry

        lax.fori_loop(0, n_chunks // 2, pair_body, 0)

    return lookup


def kernel(inputs, table):
    b, s = inputs.shape
    vocab, d = table.shape
    b_total = b * s
    info = plsc.get_sparse_core_info()
    nw = info.num_cores * info.num_subcores
    n_chunks = b_total // (nw * _CHUNK)
    idx = inputs.reshape(nw, n_chunks, _CHUNK).astype(jnp.int32)
    out = _make_lookup(b_total, vocab, d)(idx, table)
    return out.reshape(b, s, d)


# speedup vs baseline: 5.1807x; 1.8028x over previous
import functools
import math

import jax
import jax.numpy as jnp
from jax import lax
from jax.experimental import pallas as pl
from jax.experimental.pallas import tpu as pltpu
from jax.experimental.pallas import tpu_sc as plsc

_NB = 4


@functools.lru_cache(maxsize=None)
def _make_lookup(b: int, s: int, vocab: int, d: int):
    info = plsc.get_sparse_core_info()
    nc, ns, nl = info.num_cores, info.num_subcores, info.num_lanes
    nw = nc * ns
    assert d % nl == 0
    assert b % (nw * _NB) == 0
    rows_per_w = b // nw
    n_chunks = rows_per_w // _NB
    scale = jnp.float32(math.sqrt(float(d)))
    mesh = plsc.VectorSubcoreMesh(core_axis_name="c", subcore_axis_name="s")

    @functools.partial(
        pl.kernel,
        mesh=mesh,
        out_type=jax.ShapeDtypeStruct((b, s, d), jnp.float32),
        scratch_types=[
            pltpu.VMEM((rows_per_w, s), jnp.int32),
            pltpu.VMEM((_NB, s, d), jnp.float32),
            pltpu.VMEM((_NB, s, d), jnp.float32),
            pltpu.SemaphoreType.DMA,
            pltpu.SemaphoreType.DMA,
        ],
    )
    def lookup(idx_hbm, table_hbm, out_hbm, idx_v, buf0, buf1, sem0, sem1):
        wid = lax.axis_index("s") * nc + lax.axis_index("c")
        base = wid * rows_per_w
        pltpu.sync_copy(idx_hbm.at[pl.ds(base, rows_per_w)], idx_v)

        def fire(g, buf, sem):
            for r in range(_NB):
                pltpu.async_copy(
                    table_hbm.at[idx_v.at[g * _NB + r]], buf.at[r], sem
                )

        def drain(buf, sem):
            pltpu.make_async_copy(out_hbm.at[pl.ds(0, _NB)], buf, sem).wait()

        def scale_buf(buf):
            @plsc.parallel_loop(0, s, 1, unroll=2)
            def _(i):
                for r in range(_NB):
                    for j in range(d // nl):
                        sl = pl.ds(j * nl, nl)
                        buf[r, i, sl] = buf[r, i, sl] * scale

        def emit(g, buf):
            pltpu.sync_copy(buf, out_hbm.at[pl.ds(base + g * _NB, _NB)])

        fire(0, buf0, sem0)

        def pair_body(h, carry):
            g0 = 2 * h
            fire(g0 + 1, buf1, sem1)
            drain(buf0, sem0)
            scale_buf(buf0)
            emit(g0, buf0)

            @pl.when(g0 + 2 < n_chunks)
            def _():
                fire(g0 + 2, buf0, sem0)

            drain(buf1, sem1)
            scale_buf(buf1)
            emit(g0 + 1, buf1)
            return carry

        lax.fori_loop(0, n_chunks // 2, pair_body, 0)

    return lookup


def kernel(inputs, table):
    b, s = inputs.shape
    vocab, d = table.shape
    return _make_lookup(b, s, vocab, d)(inputs.astype(jnp.int32), table)


# speedup vs baseline: 5.1815x; 1.0002x over previous
import functools
import math

import jax
import jax.numpy as jnp
from jax import lax
from jax.experimental import pallas as pl
from jax.experimental.pallas import tpu as pltpu
from jax.experimental.pallas import tpu_sc as plsc

_NB = 4


@functools.lru_cache(maxsize=None)
def _make_lookup(b: int, s: int, vocab: int, d: int):
    info = plsc.get_sparse_core_info()
    nc, ns, nl = info.num_cores, info.num_subcores, info.num_lanes
    nw = nc * ns
    assert d % nl == 0
    assert b % (nw * _NB) == 0
    rows_per_w = b // nw
    n_chunks = rows_per_w // _NB
    scale = jnp.float32(math.sqrt(float(d)))
    mesh = plsc.VectorSubcoreMesh(core_axis_name="c", subcore_axis_name="s")

    @functools.partial(
        pl.kernel,
        mesh=mesh,
        out_type=jax.ShapeDtypeStruct((b, s, d), jnp.float32),
        scratch_types=[
            pltpu.VMEM((rows_per_w, s), jnp.int32),
            pltpu.VMEM((_NB, s, d), jnp.float32),
            pltpu.VMEM((_NB, s, d), jnp.float32),
            pltpu.SemaphoreType.DMA,
            pltpu.SemaphoreType.DMA,
        ],
        compiler_params=pltpu.CompilerParams(use_tc_tiling_on_sc=True),
    )
    def lookup(idx_hbm, table_hbm, out_hbm, idx_v, buf0, buf1, sem0, sem1):
        wid = lax.axis_index("s") * nc + lax.axis_index("c")
        base = wid * rows_per_w
        pltpu.sync_copy(idx_hbm.at[pl.ds(base, rows_per_w)], idx_v)

        def fire(g, buf, sem):
            for r in range(_NB):
                pltpu.async_copy(
                    table_hbm.at[idx_v.at[g * _NB + r]], buf.at[r], sem
                )

        def drain(buf, sem):
            pltpu.make_async_copy(out_hbm.at[pl.ds(0, _NB)], buf, sem).wait()

        def scale_buf(buf):
            @plsc.parallel_loop(0, s, 1, unroll=2)
            def _(i):
                for r in range(_NB):
                    for j in range(d // nl):
                        sl = pl.ds(j * nl, nl)
                        buf[r, i, sl] = buf[r, i, sl] * scale

        def emit(g, buf):
            pltpu.sync_copy(buf, out_hbm.at[pl.ds(base + g * _NB, _NB)])

        fire(0, buf0, sem0)

        def pair_body(h, carry):
            g0 = 2 * h
            fire(g0 + 1, buf1, sem1)
            drain(buf0, sem0)
            scale_buf(buf0)
            emit(g0, buf0)

            @pl.when(g0 + 2 < n_chunks)
            def _():
                fire(g0 + 2, buf0, sem0)

            drain(buf1, sem1)
            scale_buf(buf1)
            emit(g0 + 1, buf1)
            return carry

        lax.fori_loop(0, n_chunks // 2, pair_body, 0)

    return lookup


def kernel(inputs, table):
    b, s = inputs.shape
    vocab, d = table.shape
    return _make_lookup(b, s, vocab, d)(inputs.astype(jnp.int32), table)
